# baseline (speedup 1.0000x reference)
import math
import functools

import numpy as np
import jax
import jax.numpy as jnp
from jax import lax
from jax.experimental import pallas as pl
from jax.experimental.pallas import tpu as pltpu
from jax.experimental.pallas import tpu_sc as plsc

_POS_LEN = 2048
_LANES = 16
_NC = 2
_NS = 16
_NW = _NC * _NS
_CHUNK = 32


def _positional_table(length, depth):
    half = depth / 2
    positions = np.arange(length)[:, np.newaxis].astype(np.float64)
    depths = np.arange(half)[np.newaxis, :] / half
    angle_rates = 1 / 10000 ** depths
    angle_rads = positions * angle_rates
    enc = np.concatenate([np.sin(angle_rads), np.cos(angle_rads)], axis=-1)
    return jnp.asarray(enc, dtype=jnp.float32)


@functools.lru_cache(maxsize=None)
def _build(batch, seq, vocab, depth):
    n_flat = batch * seq
    bpw = n_flat // _NW
    n_chunk = bpw // _CHUNK
    wps = seq // bpw
    vregs_per_row = depth // _LANES
    scale = jnp.float32(math.sqrt(float(depth)))

    mesh = plsc.VectorSubcoreMesh(
        core_axis_name="c", subcore_axis_name="s",
        num_cores=_NC, num_subcores=_NS,
    )

    @functools.partial(
        pl.kernel,
        out_type=jax.ShapeDtypeStruct((batch, seq, depth), jnp.float32),
        mesh=mesh,
        scratch_types=[
            pltpu.VMEM((bpw,), jnp.int32),
            pltpu.VMEM((bpw, depth), jnp.float32),
            pltpu.VMEM((bpw, depth), jnp.float32),
            [pltpu.SemaphoreType.DMA] * 8,
            [pltpu.SemaphoreType.DMA] * 8,
            pltpu.SemaphoreType.DMA,
        ],
    )
    def body(x_hbm, table_hbm, pos_hbm, out_hbm, idx_v, rows_v, pos_v,
             gsems, psems, ssem):
        wid = lax.axis_index("s") * _NC + lax.axis_index("c")
        b = lax.div(wid, wps)
        s0 = lax.rem(wid, wps) * bpw
        pltpu.sync_copy(x_hbm.at[b, pl.ds(s0, bpw)], idx_v)
        gathers = [
            pltpu.async_copy(
                table_hbm.at[idx_v.at[pl.ds(c * _CHUNK, _CHUNK)]],
                rows_v.at[pl.ds(c * _CHUNK, _CHUNK)],
                gsems[c],
            )
            for c in range(n_chunk)
        ]
        pos_copies = [
            pltpu.async_copy(
                pos_hbm.at[pl.ds(s0 + c * _CHUNK, _CHUNK)],
                pos_v.at[pl.ds(c * _CHUNK, _CHUNK)],
                psems[c],
            )
            for c in range(n_chunk)
        ]

        stores = []
        for c in range(n_chunk):
            gathers[c].wait()
            pos_copies[c].wait()

            @plsc.parallel_loop(c * _CHUNK, (c + 1) * _CHUNK, unroll=4)
            def _(i):
                for j in range(vregs_per_row):
                    sl = pl.ds(j * _LANES, _LANES)
                    plsc.addupdate(pos_v.at[i, sl], rows_v[i, sl] * scale)

            stores.append(
                pltpu.async_copy(
                    pos_v.at[pl.ds(c * _CHUNK, _CHUNK)],
                    out_hbm.at[b].at[pl.ds(s0 + c * _CHUNK, _CHUNK)],
                    ssem,
                )
            )
        for st in stores:
            st.wait()

    return body


def kernel(x, table):
    batch, seq = x.shape
    vocab, depth = table.shape
    pos = _positional_table(_POS_LEN, depth)[:seq]
    body = _build(batch, seq, vocab, depth)
    return body(x, table, pos)


# speedup vs baseline: 1.0270x; 1.0270x over previous
import math
import functools

import numpy as np
import jax
import jax.numpy as jnp
from jax import lax
from jax.experimental import pallas as pl
from jax.experimental.pallas import tpu as pltpu
from jax.experimental.pallas import tpu_sc as plsc

_POS_LEN = 2048
_LANES = 16
_NC = 2
_NS = 16
_NW = _NC * _NS
_CHUNK = 64


def _positional_table(length, depth):
    half = depth / 2
    positions = np.arange(length)[:, np.newaxis].astype(np.float64)
    depths = np.arange(half)[np.newaxis, :] / half
    angle_rates = 1 / 10000 ** depths
    angle_rads = positions * angle_rates
    enc = np.concatenate([np.sin(angle_rads), np.cos(angle_rads)], axis=-1)
    return jnp.asarray(enc, dtype=jnp.float32)


@functools.lru_cache(maxsize=None)
def _build(batch, seq, vocab, depth):
    n_flat = batch * seq
    bpw = n_flat // _NW
    n_chunk = bpw // _CHUNK
    wps = seq // bpw
    vregs_per_row = depth // _LANES
    scale = jnp.float32(math.sqrt(float(depth)))

    mesh = plsc.VectorSubcoreMesh(
        core_axis_name="c", subcore_axis_name="s",
        num_cores=_NC, num_subcores=_NS,
    )

    @functools.partial(
        pl.kernel,
        out_type=jax.ShapeDtypeStruct((batch, seq, depth), jnp.float32),
        mesh=mesh,
        scratch_types=[
            pltpu.VMEM((bpw,), jnp.int32),
            pltpu.VMEM((bpw, depth), jnp.float32),
            pltpu.VMEM((bpw, depth), jnp.float32),
            [pltpu.SemaphoreType.DMA] * 4,
            pltpu.SemaphoreType.DMA,
            pltpu.SemaphoreType.DMA,
        ],
    )
    def body(x_hbm, table_hbm, pos_hbm, out_hbm, idx_v, rows_v, pos_v,
             gsems, psem, ssem):
        wid = lax.axis_index("s") * _NC + lax.axis_index("c")
        b = lax.div(wid, wps)
        s0 = lax.rem(wid, wps) * bpw
        pltpu.sync_copy(x_hbm.at[b, pl.ds(s0, bpw)], idx_v)
        gathers = [
            pltpu.async_copy(
                table_hbm.at[idx_v.at[pl.ds(c * _CHUNK, _CHUNK)]],
                rows_v.at[pl.ds(c * _CHUNK, _CHUNK)],
                gsems[c],
            )
            for c in range(n_chunk)
        ]
        pos_copy = pltpu.async_copy(pos_hbm.at[pl.ds(s0, bpw)], pos_v, psem)

        stores = []
        for c in range(n_chunk):
            gathers[c].wait()
            if c == 0:
                pos_copy.wait()

            @plsc.parallel_loop(c * _CHUNK, (c + 1) * _CHUNK, unroll=4)
            def _(i):
                for j in range(vregs_per_row):
                    sl = pl.ds(j * _LANES, _LANES)
                    plsc.addupdate(pos_v.at[i, sl], rows_v[i, sl] * scale)

            stores.append(
                pltpu.async_copy(
                    pos_v.at[pl.ds(c * _CHUNK, _CHUNK)],
                    out_hbm.at[b].at[pl.ds(s0 + c * _CHUNK, _CHUNK)],
                    ssem,
                )
            )
        for st in stores:
            st.wait()

    return body


def kernel(x, table):
    batch, seq = x.shape
    vocab, depth = table.shape
    pos = _positional_table(_POS_LEN, depth)[:seq]
    body = _build(batch, seq, vocab, depth)
    return body(x, table, pos)


# speedup vs baseline: 1.0407x; 1.0134x over previous
import math
import functools

import numpy as np
import jax
import jax.numpy as jnp
from jax import lax
from jax.experimental import pallas as pl
from jax.experimental.pallas import tpu as pltpu
from jax.experimental.pallas import tpu_sc as plsc

_POS_LEN = 2048
_LANES = 16
_NC = 2
_NS = 16
_NW = _NC * _NS
_CHUNK = 64


def _positional_table(length, depth):
    half = depth / 2
    positions = np.arange(length)[:, np.newaxis].astype(np.float64)
    depths = np.arange(half)[np.newaxis, :] / half
    angle_rates = 1 / 10000 ** depths
    angle_rads = positions * angle_rates
    enc = np.concatenate([np.sin(angle_rads), np.cos(angle_rads)], axis=-1)
    return jnp.asarray(enc, dtype=jnp.float32)


@functools.lru_cache(maxsize=None)
def _build(batch, seq, vocab, depth):
    n_flat = batch * seq
    bpw = n_flat // _NW
    n_chunk = bpw // _CHUNK
    wps = seq // bpw
    vregs_per_row = depth // _LANES
    scale = jnp.float32(math.sqrt(float(depth)))

    mesh = plsc.VectorSubcoreMesh(
        core_axis_name="c", subcore_axis_name="s",
        num_cores=_NC, num_subcores=_NS,
    )

    @functools.partial(
        pl.kernel,
        out_type=jax.ShapeDtypeStruct((batch, seq, depth), jnp.float32),
        mesh=mesh,
        scratch_types=[
            pltpu.VMEM((bpw,), jnp.int32),
            pltpu.VMEM((bpw, depth), jnp.float32),
            pltpu.VMEM((bpw, depth), jnp.float32),
            [pltpu.SemaphoreType.DMA] * 4,
            [pltpu.SemaphoreType.DMA] * 4,
            pltpu.SemaphoreType.DMA,
        ],
    )
    def body(x_hbm, table_hbm, pos_hbm, out_hbm, idx_v, rows_v, pos_v,
             gsems, psems, ssem):
        wid = lax.axis_index("s") * _NC + lax.axis_index("c")
        b = lax.div(wid, wps)
        s0 = lax.rem(wid, wps) * bpw
        pltpu.sync_copy(x_hbm.at[b, pl.ds(s0, bpw)], idx_v)
        gathers = [
            pltpu.async_copy(
                table_hbm.at[idx_v.at[pl.ds(c * _CHUNK, _CHUNK)]],
                rows_v.at[pl.ds(c * _CHUNK, _CHUNK)],
                gsems[c],
            )
            for c in range(n_chunk)
        ]
        pos_copies = [
            pltpu.async_copy(
                pos_hbm.at[pl.ds(s0 + c * _CHUNK, _CHUNK)],
                pos_v.at[pl.ds(c * _CHUNK, _CHUNK)],
                psems[c],
            )
            for c in range(n_chunk)
        ]

        stores = []
        for c in range(n_chunk):
            gathers[c].wait()
            pos_copies[c].wait()

            @plsc.parallel_loop(c * _CHUNK, (c + 1) * _CHUNK, unroll=4)
            def _(i):
                for j in range(vregs_per_row):
                    sl = pl.ds(j * _LANES, _LANES)
                    plsc.addupdate(pos_v.at[i, sl], rows_v[i, sl] * scale)

            stores.append(
                pltpu.async_copy(
                    pos_v.at[pl.ds(c * _CHUNK, _CHUNK)],
                    out_hbm.at[b].at[pl.ds(s0 + c * _CHUNK, _CHUNK)],
                    ssem,
                )
            )
        for st in stores:
            st.wait()

    return body


def kernel(x, table):
    batch, seq = x.shape
    vocab, depth = table.shape
    pos = _positional_table(_POS_LEN, depth)[:seq]
    body = _build(batch, seq, vocab, depth)
    return body(x, table, pos)


# speedup vs baseline: 1.0722x; 1.0302x over previous
import math
import functools

import numpy as np
import jax
import jax.numpy as jnp
from jax import lax
from jax.experimental import pallas as pl
from jax.experimental.pallas import tpu as pltpu
from jax.experimental.pallas import tpu_sc as plsc

_POS_LEN = 2048
_LANES = 16
_NC = 2
_NS = 16
_NW = _NC * _NS
_CHUNK = 64


def _positional_table(length, depth):
    half = depth / 2
    positions = np.arange(length)[:, np.newaxis].astype(np.float64)
    depths = np.arange(half)[np.newaxis, :] / half
    angle_rates = 1 / 10000 ** depths
    angle_rads = positions * angle_rates
    enc = np.concatenate([np.sin(angle_rads), np.cos(angle_rads)], axis=-1)
    return jnp.asarray(enc, dtype=jnp.float32)


@functools.lru_cache(maxsize=None)
def _build(batch, seq, vocab, depth):
    n_flat = batch * seq
    bpw = n_flat // _NW
    n_chunk = bpw // _CHUNK
    wps = seq // bpw
    vregs_per_row = depth // _LANES
    scale = jnp.float32(math.sqrt(float(depth)))

    mesh = plsc.VectorSubcoreMesh(
        core_axis_name="c", subcore_axis_name="s",
        num_cores=_NC, num_subcores=_NS,
    )

    @functools.partial(
        pl.kernel,
        out_type=jax.ShapeDtypeStruct((batch, seq, depth), jnp.float32),
        mesh=mesh,
        scratch_types=[
            pltpu.VMEM((bpw,), jnp.int32),
            pltpu.VMEM((bpw, depth), jnp.float32),
            pltpu.VMEM((bpw, depth), jnp.float32),
            [pltpu.SemaphoreType.DMA] * 4,
            [pltpu.SemaphoreType.DMA] * 4,
            pltpu.SemaphoreType.DMA,
        ],
    )
    def body(x_hbm, table_hbm, pos_hbm, out_hbm, idx_v, rows_v, pos_v,
             gsems, psems, ssem):
        wid = lax.axis_index("s") * _NC + lax.axis_index("c")
        b = lax.div(wid, wps)
        s0 = lax.rem(wid, wps) * bpw
        pltpu.sync_copy(x_hbm.at[b, pl.ds(s0, bpw)], idx_v)
        gathers = [
            pltpu.async_copy(
                table_hbm.at[idx_v.at[pl.ds(c * _CHUNK, _CHUNK)]],
                rows_v.at[pl.ds(c * _CHUNK, _CHUNK)],
                gsems[c],
            )
            for c in range(n_chunk)
        ]
        pos_copies = [
            pltpu.async_copy(
                pos_hbm.at[pl.ds(s0 + c * _CHUNK, _CHUNK)],
                pos_v.at[pl.ds(c * _CHUNK, _CHUNK)],
                psems[c],
            )
            for c in range(n_chunk)
        ]

        stores = []
        for c in range(n_chunk):
            gathers[c].wait()
            pos_copies[c].wait()

            @plsc.parallel_loop(c * _CHUNK, (c + 1) * _CHUNK, unroll=2)
            def _(i):
                for j in range(vregs_per_row):
                    sl = pl.ds(j * _LANES, _LANES)
                    plsc.addupdate(pos_v.at[i, sl], rows_v[i, sl] * scale)

            stores.append(
                pltpu.async_copy(
                    pos_v.at[pl.ds(c * _CHUNK, _CHUNK)],
                    out_hbm.at[b].at[pl.ds(s0 + c * _CHUNK, _CHUNK)],
                    ssem,
                )
            )
        for st in stores:
            st.wait()

    return body


def kernel(x, table):
    batch, seq = x.shape
    vocab, depth = table.shape
    pos = _positional_table(_POS_LEN, depth)[:seq]
    body = _build(batch, seq, vocab, depth)
    return body(x, table, pos)


# speedup vs baseline: 1.0799x; 1.0073x over previous
import math
import functools

import numpy as np
import jax
import jax.numpy as jnp
from jax import lax
from jax.experimental import pallas as pl
from jax.experimental.pallas import tpu as pltpu
from jax.experimental.pallas import tpu_sc as plsc

_POS_LEN = 2048
_LANES = 16
_NC = 2
_NS = 16
_NW = _NC * _NS
_CHUNK = 64


def _positional_table(length, depth):
    half = depth / 2
    positions = np.arange(length)[:, np.newaxis].astype(np.float64)
    depths = np.arange(half)[np.newaxis, :] / half
    angle_rates = 1 / 10000 ** depths
    angle_rads = positions * angle_rates
    enc = np.concatenate([np.sin(angle_rads), np.cos(angle_rads)], axis=-1)
    return jnp.asarray(enc, dtype=jnp.float32)


@functools.lru_cache(maxsize=None)
def _build(batch, seq, vocab, depth):
    n_flat = batch * seq
    bpw = n_flat // _NW
    n_chunk = bpw // _CHUNK
    wps = seq // bpw
    vregs_per_row = depth // _LANES
    scale = jnp.float32(math.sqrt(float(depth)))

    mesh = plsc.VectorSubcoreMesh(
        core_axis_name="c", subcore_axis_name="s",
        num_cores=_NC, num_subcores=_NS,
    )

    @functools.partial(
        pl.kernel,
        out_type=jax.ShapeDtypeStruct((batch, seq, depth), jnp.float32),
        mesh=mesh,
        scratch_types=[
            pltpu.VMEM((bpw,), jnp.int32),
            pltpu.VMEM((bpw, depth), jnp.float32),
            pltpu.VMEM((bpw, depth), jnp.float32),
            [pltpu.SemaphoreType.DMA] * 4,
            [pltpu.SemaphoreType.DMA] * 4,
            pltpu.SemaphoreType.DMA,
        ],
    )
    def body(x_hbm, table_hbm, pos_hbm, out_hbm, idx_v, rows_v, pos_v,
             gsems, psems, ssem):
        wid = lax.axis_index("s") * _NC + lax.axis_index("c")
        b = lax.div(wid, wps)
        s0 = lax.rem(wid, wps) * bpw
        pltpu.sync_copy(x_hbm.at[b, pl.ds(s0, bpw)], idx_v)
        gathers = [
            pltpu.async_copy(
                table_hbm.at[idx_v.at[pl.ds(c * _CHUNK, _CHUNK)]],
                rows_v.at[pl.ds(c * _CHUNK, _CHUNK)],
                gsems[c],
            )
            for c in range(n_chunk)
        ]
        pos_copies = [
            pltpu.async_copy(
                pos_hbm.at[pl.ds(s0 + c * _CHUNK, _CHUNK)],
                pos_v.at[pl.ds(c * _CHUNK, _CHUNK)],
                psems[c],
            )
            for c in range(n_chunk)
        ]

        stores = []
        for c in range(n_chunk):
            gathers[c].wait()
            pos_copies[c].wait()

            @plsc.parallel_loop(c * _CHUNK, (c + 1) * _CHUNK, unroll=1)
            def _(i):
                for j in range(vregs_per_row):
                    sl = pl.ds(j * _LANES, _LANES)
                    plsc.addupdate(pos_v.at[i, sl], rows_v[i, sl] * scale)

            stores.append(
                pltpu.async_copy(
                    pos_v.at[pl.ds(c * _CHUNK, _CHUNK)],
                    out_hbm.at[b].at[pl.ds(s0 + c * _CHUNK, _CHUNK)],
                    ssem,
                )
            )
        for st in stores:
            st.wait()

    return body


def kernel(x, table):
    batch, seq = x.shape
    vocab, depth = table.shape
    pos = _positional_table(_POS_LEN, depth)[:seq]
    body = _build(batch, seq, vocab, depth)
    return body(x, table, pos)
